# baseline (speedup 1.0000x reference)
import functools

import jax
import jax.numpy as jnp
from jax import lax
from jax.experimental import pallas as pl
from jax.experimental.pallas import tpu as pltpu
from jax.experimental.pallas import tpu_sc as plsc

N = 10000
NP = 10112
D = 128
E = 320000
NC = 2
NS = 16
NW = NC * NS
EPW = E // NW
K = 112
PADE = 304
NCH = (EPW + PADE) // K
NPH = 4
NCHP = NCH // NPH
RPT = NP // NS
DEGW = 128

_mesh = plsc.VectorSubcoreMesh(
    core_axis_name="c", subcore_axis_name="s", num_cores=NC, num_subcores=NS)


@functools.partial(
    pl.kernel,
    out_type=jax.ShapeDtypeStruct((NW * NP,), jnp.float32),
    mesh=_mesh,
    scratch_types=[
        pltpu.VMEM((NCHP, K), jnp.int32),
        pltpu.VMEM((NP,), jnp.float32),
    ],
    compiler_params=pltpu.CompilerParams(needs_layout_passes=False),
)
def _deg_kernel(dst_hbm, out_hbm, dst_v, hist_v):
    c = lax.axis_index("c")
    s = lax.axis_index("s")
    wid = s * NC + c
    zero16 = jnp.zeros((16,), jnp.float32)
    ones16 = jnp.ones((16,), jnp.float32)

    def zbody(i, carry):
        hist_v[pl.ds(i * 16, 16)] = zero16
        return carry

    lax.fori_loop(0, NP // 16, zbody, 0)

    def body(i, carry):
        j = i // (K // 16)
        l = i % (K // 16)
        idx16 = dst_v[j, pl.ds(l * 16, 16)]
        plsc.addupdate_scatter(hist_v, [idx16], ones16)
        return carry

    for ph in range(NPH):
        pltpu.sync_copy(dst_hbm.at[wid, ph], dst_v)
        lax.fori_loop(0, NCHP * (K // 16), body, 0)
    pltpu.sync_copy(hist_v, out_hbm.at[pl.ds(wid * NP, NP)])


@functools.partial(
    pl.kernel,
    out_type=jax.ShapeDtypeStruct((NC, NP, D), jnp.float32),
    mesh=_mesh,
    scratch_types=[
        pltpu.VMEM((NCHP, K), jnp.int32),
        pltpu.VMEM((NCHP, K), jnp.int32),
        pltpu.VMEM((K, D), jnp.float32),
        pltpu.VMEM((K, D), jnp.float32),
        pltpu.VMEM((K, D), jnp.float32),
        pltpu.VMEM_SHARED((NP, D), jnp.float32),
        pltpu.SemaphoreType.DMA,
        pltpu.SemaphoreType.DMA,
        pltpu.SemaphoreType.DMA,
        pltpu.SemaphoreType.DMA,
        pltpu.SemaphoreType.DMA,
        pltpu.SemaphoreType.DMA,
    ],
)
def _seg_kernel(y_hbm, src_hbm, dst_hbm, out_hbm,
                src_v, dst_v, st_0, st_1, st_2, acc_sh,
                gs_0, gs_1, gs_2, ss_0, ss_1, ss_2):
    c = lax.axis_index("c")
    s = lax.axis_index("s")
    wid = s * NC + c
    pltpu.sync_copy(y_hbm.at[pl.ds(s * RPT, RPT)],
                    acc_sh.at[pl.ds(s * RPT, RPT)])
    plsc.subcore_barrier()

    st = (st_0, st_1, st_2)
    gs = (gs_0, gs_1, gs_2)
    ss = (ss_0, ss_1, ss_2)

    def gath(j, b, jn=None):
        pltpu.async_copy(y_hbm.at[src_v.at[j]], st[b], gs[b])

    def gath_clamped(j, b):
        pltpu.async_copy(
            y_hbm.at[src_v.at[jnp.minimum(j, NCHP - 1)]], st[b], gs[b])

    def waitg(b):
        pltpu.make_async_copy(y_hbm.at[src_v.at[0]], st[b], gs[b]).wait()

    def scat(j, b):
        pltpu.async_copy(st[b], acc_sh.at[dst_v.at[j]], ss[b], add=True)

    def waits(b):
        pltpu.make_async_copy(st[b], acc_sh.at[dst_v.at[0]], ss[b]).wait()

    for ph in range(NPH):
        pltpu.sync_copy(src_hbm.at[wid, ph], src_v)
        pltpu.sync_copy(dst_hbm.at[wid, ph], dst_v)
        gath(0, 0)
        gath(1, 1)
        waitg(0)
        scat(0, 0)
        gath(2, 2)
        waitg(1)
        scat(1, 1)
        waits(0)
        gath(3, 0)

        def body(i, carry):
            for t in range(3):
                m = 3 * i + 2 + t
                b = (2 + t) % 3
                bn = (b + 2) % 3
                waitg(b)
                scat(m, b)
                waits(bn)
                gath_clamped(m + 2, bn)
            return carry

        lax.fori_loop(0, (NCHP - 2) // 3, body, 0)
        waitg(2)
        waitg(0)
        waits(1)
    plsc.subcore_barrier()
    pltpu.sync_copy(acc_sh.at[pl.ds(s * RPT, RPT)],
                    out_hbm.at[c, pl.ds(s * RPT, RPT)])


def _tca_body(x_ref, w_ref, degp_ref, y_ref, dinv_ref):
    deg = 1.0 + jnp.sum(jnp.transpose(degp_ref[...]), axis=1, keepdims=True)
    dinv = lax.rsqrt(deg)
    y_ref[...] = jnp.dot(x_ref[...], w_ref[...],
                         preferred_element_type=jnp.float32) * dinv
    dinv_ref[...] = dinv


_tca = pl.pallas_call(
    _tca_body,
    out_shape=(jax.ShapeDtypeStruct((NP, D), jnp.float32),
               jax.ShapeDtypeStruct((NP, 1), jnp.float32)),
)


def _tcb_body(y_ref, p_ref, dinv_ref, b_ref, w_ref, o_ref):
    dinv = dinv_ref[...]
    h = (p_ref[0] + p_ref[1] - y_ref[...]) * dinv + b_ref[...]
    o_ref[...] = jnp.dot(h, w_ref[...],
                         preferred_element_type=jnp.float32) * dinv


_tcb = pl.pallas_call(
    _tcb_body,
    out_shape=jax.ShapeDtypeStruct((NP, D), jnp.float32),
)


def _tcc_body(y_ref, p_ref, dinv_ref, b_ref, o_ref):
    o_ref[...] = ((p_ref[0, :N] + p_ref[1, :N] - y_ref[:N])
                  * dinv_ref[:N] + b_ref[...])


_tcc = pl.pallas_call(
    _tcc_body,
    out_shape=jax.ShapeDtypeStruct((N, D), jnp.float32),
)


def kernel(features, edge_index, W0, b0, W1, b1, W2, b2):
    dummy = (N + (jnp.arange(PADE, dtype=jnp.int32) % (NP - N)))
    dummy = jnp.broadcast_to(dummy, (NW, PADE))
    src = jnp.concatenate(
        [edge_index[0].reshape(NW, EPW), dummy],
        axis=1).reshape(NW, NPH, NCHP, K)
    dst = jnp.concatenate(
        [edge_index[1].reshape(NW, EPW), dummy],
        axis=1).reshape(NW, NPH, NCHP, K)
    xpad = jnp.concatenate(
        [features, jnp.zeros((NP - N, D), jnp.float32)], axis=0)

    degp = _deg_kernel(dst).reshape(NW, NP)
    y0, dinv = _tca(xpad, W0, degp)
    p0 = _seg_kernel(y0, src, dst)
    y1 = _tcb(y0, p0, dinv, b0.reshape(1, D), W1)
    p1 = _seg_kernel(y1, src, dst)
    y2 = _tcb(y1, p1, dinv, b1.reshape(1, D), W2)
    p2 = _seg_kernel(y2, src, dst)
    return _tcc(y2, p2, dinv, b2.reshape(1, D))


# speedup vs baseline: 1.0586x; 1.0586x over previous
import functools

import jax
import jax.numpy as jnp
from jax import lax
from jax.experimental import pallas as pl
from jax.experimental.pallas import tpu as pltpu
from jax.experimental.pallas import tpu_sc as plsc

N = 10000
NP = 10112
D = 128
E = 320000
NC = 2
NS = 16
NW = NC * NS
EPW = E // NW
K = 128
PADE = 240
NCH = (EPW + PADE) // K
NPH = 2
NCHP = NCH // NPH
RPT = NP // NS
DEGW = 128

_mesh = plsc.VectorSubcoreMesh(
    core_axis_name="c", subcore_axis_name="s", num_cores=NC, num_subcores=NS)


@functools.partial(
    pl.kernel,
    out_type=jax.ShapeDtypeStruct((NW * NP,), jnp.float32),
    mesh=_mesh,
    scratch_types=[
        pltpu.VMEM((NCH, K), jnp.int32),
        pltpu.VMEM((NP,), jnp.float32),
    ],
    compiler_params=pltpu.CompilerParams(needs_layout_passes=False),
)
def _deg_kernel(dst_hbm, out_hbm, dst_v, hist_v):
    c = lax.axis_index("c")
    s = lax.axis_index("s")
    wid = s * NC + c
    pltpu.sync_copy(dst_hbm.at[wid], dst_v)
    zero16 = jnp.zeros((16,), jnp.float32)
    ones16 = jnp.ones((16,), jnp.float32)

    def zbody(i, carry):
        hist_v[pl.ds(i * 16, 16)] = zero16
        return carry

    lax.fori_loop(0, NP // 16, zbody, 0)

    def body(i, carry):
        j = i // (K // 16)
        l = i % (K // 16)
        idx16 = dst_v[j, pl.ds(l * 16, 16)]
        plsc.addupdate_scatter(hist_v, [idx16], ones16)
        return carry

    lax.fori_loop(0, NCH * (K // 16), body, 0)
    pltpu.sync_copy(hist_v, out_hbm.at[pl.ds(wid * NP, NP)])


@functools.partial(
    pl.kernel,
    out_type=jax.ShapeDtypeStruct((NC, NP, D), jnp.float32),
    mesh=_mesh,
    scratch_types=[
        pltpu.VMEM((NCHP, K), jnp.int32),
        pltpu.VMEM((NCHP, K), jnp.int32),
        pltpu.VMEM((K, D), jnp.float32),
        pltpu.VMEM((K, D), jnp.float32),
        pltpu.VMEM_SHARED((NP, D), jnp.float32),
        pltpu.SemaphoreType.DMA,
        pltpu.SemaphoreType.DMA,
    ],
)
def _seg_kernel(y_hbm, src_hbm, dst_hbm, out_hbm,
                src_v, dst_v, st_a, st_b, acc_sh, gs_a, gs_b):
    c = lax.axis_index("c")
    s = lax.axis_index("s")
    wid = s * NC + c
    pltpu.sync_copy(y_hbm.at[pl.ds(s * RPT, RPT)],
                    acc_sh.at[pl.ds(s * RPT, RPT)])
    plsc.subcore_barrier()

    for ph in range(NPH):
        pltpu.sync_copy(src_hbm.at[wid, pl.ds(ph * NCHP, NCHP)], src_v)
        pltpu.sync_copy(dst_hbm.at[wid, pl.ds(ph * NCHP, NCHP)], dst_v)
        pltpu.async_copy(y_hbm.at[src_v.at[0]], st_a, gs_a)

        def body(i, carry):
            j = 2 * i
            pltpu.async_copy(y_hbm.at[src_v.at[j + 1]], st_b, gs_b)
            pltpu.make_async_copy(y_hbm.at[src_v.at[0]], st_a, gs_a).wait()
            pltpu.sync_copy(st_a, acc_sh.at[dst_v.at[j]], add=True)
            jn = jnp.minimum(j + 2, NCHP - 1)
            pltpu.async_copy(y_hbm.at[src_v.at[jn]], st_a, gs_a)
            pltpu.make_async_copy(y_hbm.at[src_v.at[0]], st_b, gs_b).wait()
            pltpu.sync_copy(st_b, acc_sh.at[dst_v.at[j + 1]], add=True)
            return carry

        lax.fori_loop(0, NCHP // 2, body, 0)
        pltpu.make_async_copy(y_hbm.at[src_v.at[0]], st_a, gs_a).wait()
    plsc.subcore_barrier()
    pltpu.sync_copy(acc_sh.at[pl.ds(s * RPT, RPT)],
                    out_hbm.at[c, pl.ds(s * RPT, RPT)])


def _tca_body(x_ref, w_ref, degp_ref, y_ref, dinv_ref):
    deg = 1.0 + jnp.sum(jnp.transpose(degp_ref[...]), axis=1, keepdims=True)
    dinv = lax.rsqrt(deg)
    y_ref[...] = jnp.dot(x_ref[...], w_ref[...],
                         preferred_element_type=jnp.float32) * dinv
    dinv_ref[...] = dinv


_tca = pl.pallas_call(
    _tca_body,
    out_shape=(jax.ShapeDtypeStruct((NP, D), jnp.float32),
               jax.ShapeDtypeStruct((NP, 1), jnp.float32)),
)


def _tcb_body(y_ref, p_ref, dinv_ref, b_ref, w_ref, o_ref):
    dinv = dinv_ref[...]
    h = (p_ref[0] + p_ref[1] - y_ref[...]) * dinv + b_ref[...]
    o_ref[...] = jnp.dot(h, w_ref[...],
                         preferred_element_type=jnp.float32) * dinv


_tcb = pl.pallas_call(
    _tcb_body,
    out_shape=jax.ShapeDtypeStruct((NP, D), jnp.float32),
)


def _tcc_body(y_ref, p_ref, dinv_ref, b_ref, o_ref):
    o_ref[...] = ((p_ref[0, :N] + p_ref[1, :N] - y_ref[:N])
                  * dinv_ref[:N] + b_ref[...])


_tcc = pl.pallas_call(
    _tcc_body,
    out_shape=jax.ShapeDtypeStruct((N, D), jnp.float32),
)


def kernel(features, edge_index, W0, b0, W1, b1, W2, b2):
    dummy = (N + (jnp.arange(PADE, dtype=jnp.int32) % (NP - N)))
    dummy = jnp.broadcast_to(dummy, (NW, PADE))
    src = jnp.concatenate(
        [edge_index[0].reshape(NW, EPW), dummy], axis=1).reshape(NW, NCH, K)
    dst = jnp.concatenate(
        [edge_index[1].reshape(NW, EPW), dummy], axis=1).reshape(NW, NCH, K)
    xpad = jnp.concatenate(
        [features, jnp.zeros((NP - N, D), jnp.float32)], axis=0)

    degp = _deg_kernel(dst).reshape(NW, NP)
    y0, dinv = _tca(xpad, W0, degp)
    p0 = _seg_kernel(y0, src, dst)
    y1 = _tcb(y0, p0, dinv, b0.reshape(1, D), W1)
    p1 = _seg_kernel(y1, src, dst)
    y2 = _tcb(y1, p1, dinv, b1.reshape(1, D), W2)
    p2 = _seg_kernel(y2, src, dst)
    return _tcc(y2, p2, dinv, b2.reshape(1, D))
